# Initial kernel scaffold; baseline (speedup 1.0000x reference)
import jax
import jax.numpy as jnp
from jax.experimental import pallas as pl


def kernel(z, codebook):
    raise NotImplementedError("write your pallas kernel here")


# speedup vs baseline: 1.6471x; 1.6471x over previous
import functools

import jax
import jax.numpy as jnp
from jax import lax
from jax.experimental import pallas as pl
from jax.experimental.pallas import tpu as pltpu
from jax.experimental.pallas import tpu_sc as plsc

K = 8192
C = 32
B = 8
HW = 64 * 64
N = B * HW

K_CHUNK = 512
N_CHUNKS = K // K_CHUNK


def _argmin_body(z_ref, cb_ref, idx_ref):
    zb = z_ref[0]

    def chunk(i, carry):
        best_val, best_idx = carry
        cbi = cb_ref[pl.ds(i * K_CHUNK, K_CHUNK), :]
        c2 = jnp.sum(cbi * cbi, axis=1, keepdims=True)
        mm = lax.dot_general(cbi, zb, (((1,), (0,)), ((), ())),
                             preferred_element_type=jnp.float32)
        d = c2 - 2.0 * mm
        m = jnp.min(d, axis=0, keepdims=True)
        rows = lax.broadcasted_iota(jnp.int32, (K_CHUNK, HW), 0) + i * K_CHUNK
        cand = jnp.min(jnp.where(d == m, rows, K), axis=0, keepdims=True)
        better = m < best_val
        return (jnp.where(better, m, best_val),
                jnp.where(better, cand, best_idx))

    init = (jnp.full((1, HW), jnp.inf, jnp.float32),
            jnp.zeros((1, HW), jnp.int32))
    _, best_idx = lax.fori_loop(0, N_CHUNKS, chunk, init)
    idx_ref[0] = best_idx


def _nearest_indices(z3, codebook):
    return pl.pallas_call(
        _argmin_body,
        grid=(B,),
        in_specs=[
            pl.BlockSpec((1, C, HW), lambda b: (b, 0, 0)),
            pl.BlockSpec((K, C), lambda b: (0, 0)),
        ],
        out_specs=pl.BlockSpec((1, 1, HW), lambda b: (b, 0, 0)),
        out_shape=jax.ShapeDtypeStruct((B, 1, HW), jnp.int32),
        compiler_params=pltpu.CompilerParams(
            dimension_semantics=("arbitrary",),
        ),
    )(z3, codebook)


_IDX_MINOR = 128


@functools.lru_cache(maxsize=1)
def _build_sc_gather():
    info = plsc.get_sparse_core_info()
    nw = info.num_cores * info.num_subcores
    rows_per_w = N // nw
    idx_rows = rows_per_w // _IDX_MINOR

    def body(cb_hbm, idx_hbm, out_hbm, idx_v, rows_v, sem):
        wid = lax.axis_index("s") * info.num_cores + lax.axis_index("c")
        base = wid * idx_rows
        pltpu.sync_copy(idx_hbm.at[pl.ds(base, idx_rows)], idx_v)
        copies = [
            pltpu.async_copy(cb_hbm.at[idx_v.at[j]],
                             rows_v.at[pl.ds(j * _IDX_MINOR, _IDX_MINOR)],
                             sem)
            for j in range(idx_rows)
        ]
        for cp in copies:
            cp.wait()
        pltpu.sync_copy(rows_v,
                        out_hbm.at[pl.ds(wid * rows_per_w, rows_per_w)])

    return pl.kernel(
        body,
        out_type=jax.ShapeDtypeStruct((N, C), jnp.float32),
        mesh=plsc.VectorSubcoreMesh(core_axis_name="c",
                                    subcore_axis_name="s"),
        compiler_params=pltpu.CompilerParams(use_tc_tiling_on_sc=False),
        scratch_types=[
            pltpu.VMEM((idx_rows, _IDX_MINOR), jnp.int32),
            pltpu.VMEM((rows_per_w, C), jnp.float32),
            pltpu.SemaphoreType.DMA,
        ],
    )


def kernel(z, codebook):
    z3 = z.reshape(B, C, HW)
    idx = _nearest_indices(z3, codebook)
    idx2 = idx.reshape(N // _IDX_MINOR, _IDX_MINOR)
    zq = _build_sc_gather()(codebook, idx2)
    return zq.reshape(z.shape)
